# baseline (speedup 1.0000x reference)
import jax
import jax.numpy as jnp
from jax.experimental import pallas as pl
from jax.experimental.pallas import tpu as pltpu

_N = 6400000
_LANES = 128
_ROWS = _N // _LANES
_CROWS = (12504, 12504, 12504, 12488)
_OFFS = (0, 12504, 25008, 37512)
_NCHUNK = 4


def _copy_body(in_ref, out_ref, b0, b1, b2, b3, isems, osems):
    bufs = (b0, b1, b2, b3)

    def in_copy(i):
        return pltpu.make_async_copy(
            in_ref.at[pl.ds(_OFFS[i], _CROWS[i])],
            bufs[i].at[pl.ds(0, _CROWS[i])], isems.at[i])

    def out_copy(i):
        return pltpu.make_async_copy(
            bufs[i].at[pl.ds(0, _CROWS[i])],
            out_ref.at[pl.ds(_OFFS[i], _CROWS[i])], osems.at[i])

    for i in range(_NCHUNK):
        in_copy(i).start()
    for i in range(_NCHUNK):
        in_copy(i).wait()
        out_copy(i).start()
    for i in range(_NCHUNK):
        out_copy(i).wait()


def kernel(Zj):
    x = Zj.reshape(_ROWS, _LANES)
    y = pl.pallas_call(
        _copy_body,
        out_shape=jax.ShapeDtypeStruct((_ROWS, _LANES), Zj.dtype),
        in_specs=[pl.BlockSpec(memory_space=pl.ANY)],
        out_specs=pl.BlockSpec(memory_space=pl.ANY),
        scratch_shapes=[
            pltpu.VMEM((12504, _LANES), jnp.float32),
            pltpu.VMEM((12504, _LANES), jnp.float32),
            pltpu.VMEM((12504, _LANES), jnp.float32),
            pltpu.VMEM((12488, _LANES), jnp.float32),
            pltpu.SemaphoreType.DMA((4,)),
            pltpu.SemaphoreType.DMA((4,)),
        ],
    )(x)
    return y.reshape(_N)


# speedup vs baseline: 1.0297x; 1.0297x over previous
import jax
import jax.numpy as jnp
from jax.experimental import pallas as pl
from jax.experimental.pallas import tpu as pltpu

_N = 6400000
_ROWS = 50000
_LANES = 128
_BLOCK_ROWS = 25000


def _copy_body(in_ref, out_ref):
    out_ref[...] = in_ref[...]


def kernel(Zj):
    x = Zj.reshape(_ROWS, _LANES)
    y = pl.pallas_call(
        _copy_body,
        out_shape=jax.ShapeDtypeStruct((_ROWS, _LANES), Zj.dtype),
        grid=(_ROWS // _BLOCK_ROWS,),
        in_specs=[pl.BlockSpec((_BLOCK_ROWS, _LANES), lambda i: (i, 0))],
        out_specs=pl.BlockSpec((_BLOCK_ROWS, _LANES), lambda i: (i, 0)),
    )(x)
    return y.reshape(_N)
